# Initial kernel scaffold; baseline (speedup 1.0000x reference)
import jax
import jax.numpy as jnp
from jax.experimental import pallas as pl


def kernel(h_prefix, h_item, edge_u, edge_v, W_w, W_b, a_w, a_b):
    raise NotImplementedError("write your pallas kernel here")


# speedup vs baseline: 21.8376x; 21.8376x over previous
import jax
import jax.numpy as jnp
from jax import lax
from jax.experimental import pallas as pl
from jax.experimental.pallas import tpu as pltpu
from jax.experimental.pallas import tpu_sc as plsc

N_PREFIX = 10000
N_ITEM = 10000
D = 128
NC, NS = 2, 16
NW = NC * NS
CHUNK = 128
ROWS_PER_TILE = 632
NPAD = NS * ROWS_PER_TILE
DUMMY_ROW = NPAD - 1


def _prep_body(h_ref, w_ref, a_ref, hw_ref, item_ref):
    h = h_ref[...]
    a2 = a_ref[:, D:]
    v = jnp.dot(a2, w_ref[...])
    s = jnp.sum(h * v, axis=1, keepdims=True)
    m = jnp.max(s)
    wexp = jnp.exp(s - m)
    hw_ref[...] = wexp * h
    nrm = jnp.sqrt(jnp.sum(h * h, axis=1, keepdims=True))
    item_ref[...] = h / jnp.maximum(nrm, 1e-12)


def _finish_body(acc2_ref, out_ref):
    acc = acc2_ref[0, :N_PREFIX, :] + acc2_ref[1, :N_PREFIX, :]
    nrm = jnp.sqrt(jnp.sum(acc * acc, axis=1, keepdims=True))
    out_ref[...] = acc / jnp.maximum(nrm, 1e-12)


def _make_scatter(chunks_per_tile):
    def _scatter_body(eu_hbm, ev_hbm, hw_hbm, zeros_hbm, out_hbm,
                      idx_u, idx_v, rows, acc_sh, gsem):
        cid = lax.axis_index("c")
        sid = lax.axis_index("s")
        r0 = sid * ROWS_PER_TILE
        pltpu.sync_copy(zeros_hbm, acc_sh.at[pl.ds(r0, ROWS_PER_TILE)])
        pltpu.sync_copy(eu_hbm.at[cid, sid], idx_u)
        pltpu.sync_copy(ev_hbm.at[cid, sid], idx_v)
        plsc.subcore_barrier()

        def chunk(j, carry):
            pltpu.async_copy(hw_hbm.at[idx_v.at[j]], rows, gsem).wait()
            pltpu.sync_copy(rows, acc_sh.at[idx_u.at[j]], add=True)
            return carry

        lax.fori_loop(0, chunks_per_tile, chunk, 0)

        plsc.subcore_barrier()
        pltpu.sync_copy(acc_sh.at[pl.ds(r0, ROWS_PER_TILE)],
                        out_hbm.at[cid, pl.ds(r0, ROWS_PER_TILE)])

    return pl.kernel(
        _scatter_body,
        out_type=jax.ShapeDtypeStruct((NC, NPAD, D), jnp.float32),
        mesh=plsc.VectorSubcoreMesh(core_axis_name="c", subcore_axis_name="s"),
        scratch_types=[
            pltpu.VMEM((chunks_per_tile, CHUNK), jnp.int32),
            pltpu.VMEM((chunks_per_tile, CHUNK), jnp.int32),
            pltpu.VMEM((CHUNK, D), jnp.float32),
            pltpu.VMEM_SHARED((NPAD, D), jnp.float32),
            pltpu.SemaphoreType.DMA,
        ],
    )


def kernel(h_prefix, h_item, edge_u, edge_v, W_w, W_b, a_w, a_b):
    eu = edge_u.astype(jnp.int32)
    ev = edge_v.astype(jnp.int32)
    e = eu.shape[0]
    per_round = NW * CHUNK
    chunks_per_tile = (e + per_round - 1) // per_round
    epad = chunks_per_tile * per_round
    pad = epad - e
    eu_p = jnp.concatenate([eu, jnp.full((pad,), DUMMY_ROW, jnp.int32)])
    ev_p = jnp.concatenate([ev, jnp.zeros((pad,), jnp.int32)])
    eu_r = eu_p.reshape(NC, NS, chunks_per_tile, CHUNK)
    ev_r = ev_p.reshape(NC, NS, chunks_per_tile, CHUNK)
    zeros = jnp.zeros((ROWS_PER_TILE, D), jnp.float32)

    hw, item_out = pl.pallas_call(
        _prep_body,
        out_shape=[
            jax.ShapeDtypeStruct((N_ITEM, D), jnp.float32),
            jax.ShapeDtypeStruct((N_ITEM, D), jnp.float32),
        ],
    )(h_item, W_w, a_w)

    acc2 = _make_scatter(chunks_per_tile)(eu_r, ev_r, hw, zeros)

    prefix_out = pl.pallas_call(
        _finish_body,
        out_shape=jax.ShapeDtypeStruct((N_PREFIX, D), jnp.float32),
    )(acc2)
    return prefix_out, item_out
